# Initial kernel scaffold; baseline (speedup 1.0000x reference)
import jax
import jax.numpy as jnp
from jax.experimental import pallas as pl


def kernel(x_gene, x_miRNA, x_TO, e_gene_gene, e_gene_pathway, e_gene_mirna, e_gene_TO, e_mirna_mirna, e_mirna_TO, e_TO_TO, params):
    raise NotImplementedError("write your pallas kernel here")


# speedup vs baseline: 1.3258x; 1.3258x over previous
import jax
import jax.numpy as jnp
from jax.experimental import pallas as pl


def _gat(x, edge_index, p, heads, out_ch):
    N = x.shape[0]
    src, dst = edge_index[0], edge_index[1]
    h = (x @ p["W"]).reshape(N, heads, out_ch)
    a_src = (h * p["att_src"][None, :, :]).sum(-1)
    a_dst = (h * p["att_dst"][None, :, :]).sum(-1)
    M = jax.nn.leaky_relu(a_src.max(0) + a_dst.max(0), 0.2)
    e = jax.nn.leaky_relu(a_src[src] + a_dst[dst], 0.2)
    w = jnp.exp(e - M[None, :])
    w_self = jnp.exp(jax.nn.leaky_relu(a_src + a_dst, 0.2) - M[None, :])
    denom = jax.ops.segment_sum(w, dst, num_segments=N) + w_self
    num = jax.ops.segment_sum(h[src] * w[:, :, None], dst, num_segments=N)
    num = num + h * w_self[:, :, None]
    out = num / (denom[:, :, None] + 1e-16)
    return out.reshape(N, heads * out_ch) + p["bias"]


def kernel(x_gene, x_miRNA, x_TO, e_gene_gene, e_gene_pathway, e_gene_mirna,
           e_gene_TO, e_mirna_mirna, e_mirna_TO, e_TO_TO, params):
    gx = jax.nn.elu(_gat(x_gene, e_gene_gene, params["gene_conv1"], 4, 256))
    gx = _gat(gx, e_gene_gene, params["gene_conv2"], 1, 256)
    gx = jax.nn.elu(_gat(gx, e_gene_pathway, params["gene_pathway_conv"], 1, 256))
    gx = jax.nn.elu(_gat(gx, e_gene_mirna, params["gene_miRNA_conv"], 1, 256))
    gx = jax.nn.elu(_gat(gx, e_gene_TO, params["gene_TO_conv"], 1, 256))
    gx = gx @ params["fc_W"] + params["fc_b"]

    mx = jax.nn.elu(_gat(x_miRNA, e_mirna_mirna, params["miRNA_conv1"], 8, 128))
    mx = _gat(mx, e_mirna_mirna, params["miRNA_conv2"], 1, 128)
    mx = jax.nn.elu(_gat(mx, e_gene_mirna, params["miRNA_gene_conv"], 1, 128))
    mx = _gat(mx, e_mirna_TO, params["miRNA_TO_conv"], 1, 128)

    tx = jax.nn.elu(_gat(x_TO, e_TO_TO, params["TO_conv1"], 4, 256))
    tx = _gat(tx, e_TO_TO, params["TO_conv2"], 1, 128)
    tx = jax.nn.elu(_gat(tx, e_gene_TO, params["TO_gene_conv"], 1, 128))
    tx = jax.nn.elu(_gat(tx, e_mirna_TO, params["TO_miRNA_conv"], 1, 128))
    return (gx, mx, tx)


# speedup vs baseline: 17.1243x; 12.9161x over previous
import functools

import jax
import jax.numpy as jnp
from jax import lax
from jax.experimental import pallas as pl
from jax.experimental.pallas import tpu as pltpu
from jax.experimental.pallas import tpu_sc as plsc

f32 = jnp.float32
i32 = jnp.int32

NC, NS, L = 2, 16, 16
NW = NC * NS
N = 10000
NPAD = 10240
E = 160000
EFULL = E + N
EPAD = NW * 5376
EROWS = EPAD // 128
BN = 1024


def _matmul(x, W, bias=None):
    Cin, C = W.shape
    NB = C // 128
    Wb = jnp.transpose(W.reshape(Cin, NB, 128), (1, 0, 2))

    if bias is None:
        def body(x_ref, w_ref, o_ref):
            o_ref[0] = jnp.dot(x_ref[...], w_ref[0], preferred_element_type=f32)
        in_specs = [
            pl.BlockSpec((BN, Cin), lambda cb, i: (i, 0)),
            pl.BlockSpec((1, Cin, 128), lambda cb, i: (cb, 0, 0)),
        ]
        args = (x, Wb)
    else:
        bb = bias.reshape(NB, 1, 128)

        def body(x_ref, w_ref, b_ref, o_ref):
            o_ref[0] = (jnp.dot(x_ref[...], w_ref[0], preferred_element_type=f32)
                        + b_ref[0])
        in_specs = [
            pl.BlockSpec((BN, Cin), lambda cb, i: (i, 0)),
            pl.BlockSpec((1, Cin, 128), lambda cb, i: (cb, 0, 0)),
            pl.BlockSpec((1, 1, 128), lambda cb, i: (cb, 0, 0)),
        ]
        args = (x, Wb, bb)

    return pl.pallas_call(
        body,
        grid=(NB, NPAD // BN),
        in_specs=in_specs,
        out_specs=pl.BlockSpec((1, BN, 128), lambda cb, i: (cb, i, 0)),
        out_shape=jax.ShapeDtypeStruct((NB, NPAD, 128), f32),
    )(*args)


def _att_scalars(h, att_src, att_dst, H):
    NB = h.shape[0]
    bpb = NB // H
    asv = att_src.reshape(NB, 1, 128)
    adv = att_dst.reshape(NB, 1, 128)

    def body(h_ref, as_ref, ad_ref, os_ref, od_ref):
        hb = h_ref[...]
        ts = (hb * as_ref[...]).sum(-1)
        td = (hb * ad_ref[...]).sum(-1)
        os_ref[...] = jnp.concatenate(
            [ts[hd * bpb:(hd + 1) * bpb].sum(0, keepdims=True) for hd in range(H)], 0)
        od_ref[...] = jnp.concatenate(
            [td[hd * bpb:(hd + 1) * bpb].sum(0, keepdims=True) for hd in range(H)], 0)

    return pl.pallas_call(
        body,
        grid=(NPAD // BN,),
        in_specs=[
            pl.BlockSpec((NB, BN, 128), lambda i: (0, i, 0)),
            pl.BlockSpec((NB, 1, 128), lambda i: (0, 0, 0)),
            pl.BlockSpec((NB, 1, 128), lambda i: (0, 0, 0)),
        ],
        out_specs=[
            pl.BlockSpec((H, BN), lambda i: (0, i)),
            pl.BlockSpec((H, BN), lambda i: (0, i)),
        ],
        out_shape=[
            jax.ShapeDtypeStruct((H, NPAD), f32),
            jax.ShapeDtypeStruct((H, NPAD), f32),
        ],
    )(h, asv, adv)


def _softmax_shift(a_srcT, a_dstT, H):
    def body(as_ref, ad_ref, m_ref):
        M = (jnp.max(as_ref[...], axis=1, keepdims=True)
             + jnp.max(ad_ref[...], axis=1, keepdims=True))
        M = jnp.maximum(M, 0.2 * M)
        m_ref[...] = jnp.broadcast_to(M, (H, 128))

    return pl.pallas_call(
        body,
        in_specs=[pl.BlockSpec((H, NPAD), lambda: (0, 0)),
                  pl.BlockSpec((H, NPAD), lambda: (0, 0))],
        out_specs=pl.BlockSpec((H, 128), lambda: (0, 0)),
        out_shape=jax.ShapeDtypeStruct((H, 128), f32),
    )(a_srcT, a_dstT)


def _epilogue(h, num, den, bias, H, F, act):
    NB = h.shape[0]
    P = num.shape[0]
    C = NB * 128
    bpb = F // 128
    bb = bias.reshape(1, C)

    def body(h_ref, num_ref, den_ref, b_ref, o_ref):
        i = pl.program_id(0)
        valid = (i * BN + lax.broadcasted_iota(i32, (BN, 128), 0)) < N
        dsum = den_ref[0] + den_ref[1]
        for cb in range(NB):
            hd = cb // bpb
            nm = num_ref[0, cb]
            for p in range(1, P):
                nm = nm + num_ref[p, cb]
            dn = dsum[:, hd:hd + 1] + 1e-16
            o = nm / dn + b_ref[:, cb * 128:(cb + 1) * 128]
            if act:
                o = jnp.where(o > 0, o, jnp.exp(jnp.minimum(o, 0.0)) - 1.0)
            o_ref[:, cb * 128:(cb + 1) * 128] = jnp.where(valid, o, 0.0)

    return pl.pallas_call(
        body,
        grid=(NPAD // BN,),
        in_specs=[
            pl.BlockSpec((NB, BN, 128), lambda i: (0, i, 0)),
            pl.BlockSpec((P, NB, BN, 128), lambda i: (0, 0, i, 0)),
            pl.BlockSpec((NC, BN, H), lambda i: (0, i, 0)),
            pl.BlockSpec((1, C), lambda i: (0, 0)),
        ],
        out_specs=pl.BlockSpec((BN, C), lambda i: (i, 0)),
        out_shape=jax.ShapeDtypeStruct((NPAD, C), f32),
    )(h, num, den, bb)


_MESH = plsc.VectorSubcoreMesh(core_axis_name="c", subcore_axis_name="s",
                               num_cores=NC, num_subcores=NS)

EW = EPAD // NW
ER = EW // 128
ZR = NPAD // NS


def _pass_a(a_srcF, a_dstF, m_b, srcR, dstR, zrow, hb):
    @functools.partial(
        pl.kernel,
        mesh=_MESH,
        compiler_params=pltpu.CompilerParams(use_tc_tiling_on_sc=False,
                                             needs_layout_passes=False),
        out_type=[jax.ShapeDtypeStruct((NW * hb * EW,), f32),
                  jax.ShapeDtypeStruct((NC * hb * NPAD,), f32)],
        scratch_types=[
            pltpu.VMEM((hb * NPAD,), f32),
            pltpu.VMEM((hb * NPAD,), f32),
            pltpu.VMEM((hb, 128), f32),
            pltpu.VMEM((ER, 128), i32),
            pltpu.VMEM((ER, 128), i32),
            pltpu.VMEM((hb * EW,), f32),
            pltpu.VMEM((128,), i32),
            pltpu.VMEM((128,), i32),
            pltpu.VMEM_SHARED((hb * NPAD,), f32),
        ])
    def k(asrc_hbm, adst_hbm, m_hbm, src_hbm, dst_hbm, z_hbm,
          w_hbm, den_hbm,
          asrc_v, adst_v, m_v, src_v, dst_v, wT_v, eix0_v, eix1_v, acc_sh):
        c = lax.axis_index("c")
        s = lax.axis_index("s")
        wid = c * NS + s
        eixs = [eix0_v, eix1_v]
        pltpu.sync_copy(asrc_hbm, asrc_v)
        pltpu.sync_copy(adst_hbm, adst_v)
        pltpu.sync_copy(m_hbm, m_v)
        pltpu.sync_copy(src_hbm.at[wid], src_v)
        pltpu.sync_copy(dst_hbm.at[wid], dst_v)
        for h in range(hb):
            pltpu.sync_copy(z_hbm, acc_sh.at[pl.ds(h * NPAD + s * ZR, ZR)])
        plsc.subcore_barrier()

        def sub(kk, _):
            for gg in range(8):
                sv = src_v[kk, pl.ds(gg * L, L)]
                dv = dst_v[kk, pl.ds(gg * L, L)]
                for h in range(hb):
                    av = plsc.load_gather(asrc_v, [sv + h * NPAD])
                    bv = plsc.load_gather(adst_v, [dv + h * NPAD])
                    e = av + bv
                    e = jnp.maximum(e, 0.2 * e)
                    w = jnp.exp(e - m_v[h, pl.ds(0, L)])
                    wT_v[pl.ds(h * EW + kk * 128 + gg * L, L)] = w
                    eixs[h][pl.ds(gg * L, L)] = dv + h * NPAD
            for h in range(hb):
                pltpu.sync_copy(wT_v.at[pl.ds(h * EW + kk * 128, 128)],
                                acc_sh.at[eixs[h]], add=True)
            return 0

        lax.fori_loop(0, ER, sub, 0)
        pltpu.sync_copy(wT_v, w_hbm.at[pl.ds(wid * hb * EW, hb * EW)])

        plsc.subcore_barrier()
        for h in range(hb):
            pltpu.sync_copy(
                acc_sh.at[pl.ds(h * NPAD + s * ZR, ZR)],
                den_hbm.at[pl.ds((c * hb + h) * NPAD + s * ZR, ZR)])

    return k(a_srcF, a_dstF, m_b, srcR, dstR, zrow)


def _pass_b(h_flat, w, srcR, dstR, zrow, F, NB, H):
    bpb = F // 128
    P = 1 if NB > 1 else NC
    ncb = NB // NC if NB > 1 else 1
    nch = 2 if NB > 1 else 1

    @functools.partial(
        pl.kernel,
        mesh=_MESH,
        compiler_params=pltpu.CompilerParams(use_tc_tiling_on_sc=False,
                                             needs_layout_passes=False),
        out_type=jax.ShapeDtypeStruct((P, NB, NPAD, 128), f32),
        scratch_types=[
            pltpu.VMEM((ER, 128), i32),
            pltpu.VMEM((ER, 128), i32),
            pltpu.VMEM((EW,), f32),
            pltpu.VMEM((128,), i32),
            pltpu.VMEM((128, 128), f32),
            pltpu.VMEM_SHARED((NPAD, 128), f32),
            pltpu.SemaphoreType.DMA,
        ])
    def k(h_hbm, w_hbm, src_hbm, dst_hbm, z_hbm, num_hbm,
          src_v, dst_v, w_v, idx_v, rows_v, acc_sh, sem):
        c = lax.axis_index("c")
        s = lax.axis_index("s")

        for i in range(ncb):
            cb = i * NC + c if NB > 1 else 0
            hd = cb // bpb
            cbN = cb * NPAD
            pltpu.sync_copy(z_hbm, acc_sh.at[pl.ds(s * ZR, ZR)])
            plsc.subcore_barrier()

            for mm in range(nch):
                m = nch * s + mm if NB > 1 else c * NS + s
                pltpu.sync_copy(src_hbm.at[m], src_v)
                pltpu.sync_copy(dst_hbm.at[m], dst_v)
                pltpu.sync_copy(w_hbm.at[pl.ds((m * H + hd) * EW, EW)], w_v)

                def sub(kk, _):
                    def mkidx(t, _):
                        idx_v[pl.ds(t * L, L)] = src_v[kk, pl.ds(t * L, L)] + cbN
                        return 0
                    lax.fori_loop(0, 8, mkidx, 0)
                    pltpu.async_copy(h_hbm.at[idx_v], rows_v, sem).wait()

                    def scale(j, _):
                        ws = plsc.load_gather(w_v, [jnp.full((L,), 0, i32) + (kk * 128 + j)])
                        for t in range(8):
                            sl = pl.ds(t * L, L)
                            rows_v[j, sl] = rows_v[j, sl] * ws
                        return 0
                    lax.fori_loop(0, 128, scale, 0)
                    pltpu.sync_copy(rows_v, acc_sh.at[dst_v.at[kk]], add=True)
                    return 0

                lax.fori_loop(0, ER, sub, 0)

            plsc.subcore_barrier()
            p = 0 if NB > 1 else c
            pltpu.sync_copy(acc_sh.at[pl.ds(s * ZR, ZR)],
                            num_hbm.at[p, cb, pl.ds(s * ZR, ZR)])

    return k(h_flat, w, srcR, dstR, zrow)


def _prep_edges(e):
    pad = EPAD - EFULL
    loop = jnp.arange(N, dtype=i32)
    fill = jnp.arange(pad, dtype=i32)
    src = jnp.concatenate([e[0], loop, fill % N])
    dst = jnp.concatenate([e[1], loop, N + (fill % 224)])
    return src.reshape(NW, ER, 128), dst.reshape(NW, ER, 128)


def _gat(x, epack, p, H, F, act):
    C = H * F
    NB = C // 128
    srcR, dstR = epack
    h = _matmul(x, p["W"])
    a_srcT, a_dstT = _att_scalars(h, p["att_src"], p["att_dst"], H)
    m_b = _softmax_shift(a_srcT, a_dstT, H)
    hbs = 1 if H == 1 else 2
    zrow_a = jnp.zeros((ZR,), f32)
    wps, dens = [], []
    for h0 in range(0, H, hbs):
        w_i, d_i = _pass_a(a_srcT[h0:h0 + hbs].reshape(-1),
                           a_dstT[h0:h0 + hbs].reshape(-1),
                           m_b[h0:h0 + hbs], srcR, dstR, zrow_a, hbs)
        wps.append(w_i.reshape(NW, hbs, EW))
        dens.append(d_i.reshape(NC, hbs, NPAD))
    w = wps[0] if len(wps) == 1 else jnp.concatenate(wps, 1)
    den = dens[0] if len(dens) == 1 else jnp.concatenate(dens, 1)
    den = jnp.transpose(den, (0, 2, 1))
    zrow_b = jnp.zeros((ZR, 128), f32)
    num = _pass_b(h.reshape(NB * NPAD, 128), w.reshape(-1), srcR, dstR,
                  zrow_b, F, NB, H)
    return _epilogue(h, num, den, p["bias"], H, F, act)


def kernel(x_gene, x_miRNA, x_TO, e_gene_gene, e_gene_pathway, e_gene_mirna,
           e_gene_TO, e_mirna_mirna, e_mirna_TO, e_TO_TO, params):
    egg = _prep_edges(e_gene_gene)
    egp = _prep_edges(e_gene_pathway)
    egm = _prep_edges(e_gene_mirna)
    egt = _prep_edges(e_gene_TO)
    emm = _prep_edges(e_mirna_mirna)
    emt = _prep_edges(e_mirna_TO)
    ett = _prep_edges(e_TO_TO)

    zp = ((0, NPAD - N), (0, 0))
    x_gene = jnp.pad(x_gene, zp)
    x_miRNA = jnp.pad(x_miRNA, zp)
    x_TO = jnp.pad(x_TO, zp)

    gx = _gat(x_gene, egg, params["gene_conv1"], 4, 256, True)
    gx = _gat(gx, egg, params["gene_conv2"], 1, 256, False)
    gx = _gat(gx, egp, params["gene_pathway_conv"], 1, 256, True)
    gx = _gat(gx, egm, params["gene_miRNA_conv"], 1, 256, True)
    gx = _gat(gx, egt, params["gene_TO_conv"], 1, 256, True)
    gx = _matmul(gx, params["fc_W"], params["fc_b"]).reshape(NPAD, 128)

    mx = _gat(x_miRNA, emm, params["miRNA_conv1"], 8, 128, True)
    mx = _gat(mx, emm, params["miRNA_conv2"], 1, 128, False)
    mx = _gat(mx, egm, params["miRNA_gene_conv"], 1, 128, True)
    mx = _gat(mx, emt, params["miRNA_TO_conv"], 1, 128, False)

    tx = _gat(x_TO, ett, params["TO_conv1"], 4, 256, True)
    tx = _gat(tx, ett, params["TO_conv2"], 1, 128, False)
    tx = _gat(tx, egt, params["TO_gene_conv"], 1, 128, True)
    tx = _gat(tx, emt, params["TO_miRNA_conv"], 1, 128, True)
    return (gx[:N], mx[:N], tx[:N])


# speedup vs baseline: 25.8733x; 1.5109x over previous
import functools

import jax
import jax.numpy as jnp
from jax import lax
from jax.experimental import pallas as pl
from jax.experimental.pallas import tpu as pltpu
from jax.experimental.pallas import tpu_sc as plsc

f32 = jnp.float32
i32 = jnp.int32

NC, NS, L = 2, 16, 16
NW = NC * NS
N = 10000
NPAD = 10240
E = 160000
EFULL = E + N
EPAD = NW * 5376
EROWS = EPAD // 128
BN = 1024


def _matmul(x, W, bias=None):
    Cin, C = W.shape
    NB = C // 128
    Wb = jnp.transpose(W.reshape(Cin, NB, 128), (1, 0, 2))

    if bias is None:
        def body(x_ref, w_ref, o_ref):
            o_ref[0] = jnp.dot(x_ref[...], w_ref[0], preferred_element_type=f32)
        in_specs = [
            pl.BlockSpec((BN, Cin), lambda cb, i: (i, 0)),
            pl.BlockSpec((1, Cin, 128), lambda cb, i: (cb, 0, 0)),
        ]
        args = (x, Wb)
    else:
        bb = bias.reshape(NB, 1, 128)

        def body(x_ref, w_ref, b_ref, o_ref):
            o_ref[0] = (jnp.dot(x_ref[...], w_ref[0], preferred_element_type=f32)
                        + b_ref[0])
        in_specs = [
            pl.BlockSpec((BN, Cin), lambda cb, i: (i, 0)),
            pl.BlockSpec((1, Cin, 128), lambda cb, i: (cb, 0, 0)),
            pl.BlockSpec((1, 1, 128), lambda cb, i: (cb, 0, 0)),
        ]
        args = (x, Wb, bb)

    return pl.pallas_call(
        body,
        grid=(NB, NPAD // BN),
        in_specs=in_specs,
        out_specs=pl.BlockSpec((1, BN, 128), lambda cb, i: (cb, i, 0)),
        out_shape=jax.ShapeDtypeStruct((NB, NPAD, 128), f32),
    )(*args)


def _att_scalars(h, att_src, att_dst, H):
    NB = h.shape[0]
    bpb = NB // H
    asv = att_src.reshape(NB, 1, 128)
    adv = att_dst.reshape(NB, 1, 128)

    def body(h_ref, as_ref, ad_ref, os_ref, od_ref):
        hb = h_ref[...]
        ts = (hb * as_ref[...]).sum(-1)
        td = (hb * ad_ref[...]).sum(-1)
        os_ref[...] = jnp.concatenate(
            [ts[hd * bpb:(hd + 1) * bpb].sum(0, keepdims=True) for hd in range(H)], 0)
        od_ref[...] = jnp.concatenate(
            [td[hd * bpb:(hd + 1) * bpb].sum(0, keepdims=True) for hd in range(H)], 0)

    return pl.pallas_call(
        body,
        grid=(NPAD // BN,),
        in_specs=[
            pl.BlockSpec((NB, BN, 128), lambda i: (0, i, 0)),
            pl.BlockSpec((NB, 1, 128), lambda i: (0, 0, 0)),
            pl.BlockSpec((NB, 1, 128), lambda i: (0, 0, 0)),
        ],
        out_specs=[
            pl.BlockSpec((H, BN), lambda i: (0, i)),
            pl.BlockSpec((H, BN), lambda i: (0, i)),
        ],
        out_shape=[
            jax.ShapeDtypeStruct((H, NPAD), f32),
            jax.ShapeDtypeStruct((H, NPAD), f32),
        ],
    )(h, asv, adv)


def _softmax_shift(a_srcT, a_dstT, H):
    def body(as_ref, ad_ref, m_ref):
        M = (jnp.max(as_ref[...], axis=1, keepdims=True)
             + jnp.max(ad_ref[...], axis=1, keepdims=True))
        M = jnp.maximum(M, 0.2 * M)
        m_ref[...] = jnp.broadcast_to(M, (H, 128))

    return pl.pallas_call(
        body,
        in_specs=[pl.BlockSpec((H, NPAD), lambda: (0, 0)),
                  pl.BlockSpec((H, NPAD), lambda: (0, 0))],
        out_specs=pl.BlockSpec((H, 128), lambda: (0, 0)),
        out_shape=jax.ShapeDtypeStruct((H, 128), f32),
    )(a_srcT, a_dstT)


def _epilogue(h, num, den, bias, H, F, act):
    NB = h.shape[0]
    P = num.shape[0]
    C = NB * 128
    bpb = F // 128
    bb = bias.reshape(1, C)

    def body(h_ref, num_ref, den_ref, b_ref, o_ref):
        i = pl.program_id(0)
        valid = (i * BN + lax.broadcasted_iota(i32, (BN, 128), 0)) < N
        dsum = den_ref[0] + den_ref[1]
        for cb in range(NB):
            hd = cb // bpb
            nm = num_ref[0, cb]
            for p in range(1, P):
                nm = nm + num_ref[p, cb]
            dn = dsum[:, hd:hd + 1] + 1e-16
            o = nm / dn + b_ref[:, cb * 128:(cb + 1) * 128]
            if act:
                o = jnp.where(o > 0, o, jnp.exp(jnp.minimum(o, 0.0)) - 1.0)
            o_ref[:, cb * 128:(cb + 1) * 128] = jnp.where(valid, o, 0.0)

    return pl.pallas_call(
        body,
        grid=(NPAD // BN,),
        in_specs=[
            pl.BlockSpec((NB, BN, 128), lambda i: (0, i, 0)),
            pl.BlockSpec((P, NB, BN, 128), lambda i: (0, 0, i, 0)),
            pl.BlockSpec((NC, BN, H), lambda i: (0, i, 0)),
            pl.BlockSpec((1, C), lambda i: (0, 0)),
        ],
        out_specs=pl.BlockSpec((BN, C), lambda i: (i, 0)),
        out_shape=jax.ShapeDtypeStruct((NPAD, C), f32),
    )(h, num, den, bb)


_MESH = plsc.VectorSubcoreMesh(core_axis_name="c", subcore_axis_name="s",
                               num_cores=NC, num_subcores=NS)

EW = EPAD // NW
ER = EW // 128
ZR = NPAD // NS


def _pass_a(a_srcF, a_dstF, m_b, srcR, dstR, zrow, hb):
    @functools.partial(
        pl.kernel,
        mesh=_MESH,
        compiler_params=pltpu.CompilerParams(use_tc_tiling_on_sc=False,
                                             needs_layout_passes=False),
        out_type=[jax.ShapeDtypeStruct((NW * hb * EW,), f32),
                  jax.ShapeDtypeStruct((NC * hb * NPAD,), f32)],
        scratch_types=[
            pltpu.VMEM((hb * NPAD,), f32),
            pltpu.VMEM((hb * NPAD,), f32),
            pltpu.VMEM((hb, 128), f32),
            pltpu.VMEM((ER, 128), i32),
            pltpu.VMEM((ER, 128), i32),
            pltpu.VMEM((hb * EW,), f32),
            pltpu.VMEM((128,), i32),
            pltpu.VMEM((128,), i32),
            pltpu.VMEM_SHARED((hb * NPAD,), f32),
        ])
    def k(asrc_hbm, adst_hbm, m_hbm, src_hbm, dst_hbm, z_hbm,
          w_hbm, den_hbm,
          asrc_v, adst_v, m_v, src_v, dst_v, wT_v, eix0_v, eix1_v, acc_sh):
        c = lax.axis_index("c")
        s = lax.axis_index("s")
        wid = c * NS + s
        eixs = [eix0_v, eix1_v]
        pltpu.sync_copy(asrc_hbm, asrc_v)
        pltpu.sync_copy(adst_hbm, adst_v)
        pltpu.sync_copy(m_hbm, m_v)
        pltpu.sync_copy(src_hbm.at[wid], src_v)
        pltpu.sync_copy(dst_hbm.at[wid], dst_v)
        for h in range(hb):
            pltpu.sync_copy(z_hbm, acc_sh.at[pl.ds(h * NPAD + s * ZR, ZR)])
        plsc.subcore_barrier()

        def sub(kk, _):
            for gg in range(8):
                sv = src_v[kk, pl.ds(gg * L, L)]
                dv = dst_v[kk, pl.ds(gg * L, L)]
                for h in range(hb):
                    av = plsc.load_gather(asrc_v, [sv + h * NPAD])
                    bv = plsc.load_gather(adst_v, [dv + h * NPAD])
                    e = av + bv
                    e = jnp.maximum(e, 0.2 * e)
                    w = jnp.exp(e - m_v[h, pl.ds(0, L)])
                    wT_v[pl.ds(h * EW + kk * 128 + gg * L, L)] = w
                    eixs[h][pl.ds(gg * L, L)] = dv + h * NPAD
            for h in range(hb):
                pltpu.sync_copy(wT_v.at[pl.ds(h * EW + kk * 128, 128)],
                                acc_sh.at[eixs[h]], add=True)
            return 0

        lax.fori_loop(0, ER, sub, 0)
        pltpu.sync_copy(wT_v, w_hbm.at[pl.ds(wid * hb * EW, hb * EW)])

        plsc.subcore_barrier()
        for h in range(hb):
            pltpu.sync_copy(
                acc_sh.at[pl.ds(h * NPAD + s * ZR, ZR)],
                den_hbm.at[pl.ds((c * hb + h) * NPAD + s * ZR, ZR)])

    return k(a_srcF, a_dstF, m_b, srcR, dstR, zrow)


def _pass_b(h_flat, w, srcR, dstR, zrow, F, NB, H):
    bpb = F // 128
    P = 1 if NB > 1 else NC
    ncb = NB // NC if NB > 1 else 1
    nch = 2 if NB > 1 else 1

    @functools.partial(
        pl.kernel,
        mesh=_MESH,
        compiler_params=pltpu.CompilerParams(use_tc_tiling_on_sc=False,
                                             needs_layout_passes=False),
        out_type=jax.ShapeDtypeStruct((P, NB, NPAD, 128), f32),
        scratch_types=[
            pltpu.VMEM((ER, 128), i32),
            pltpu.VMEM((ER, 128), i32),
            pltpu.VMEM((EW,), f32),
            pltpu.VMEM((128,), i32),
            pltpu.VMEM((128,), i32),
            pltpu.VMEM((128, 128), f32),
            pltpu.VMEM((128, 128), f32),
            pltpu.VMEM_SHARED((NPAD, 128), f32),
            pltpu.SemaphoreType.DMA,
            pltpu.SemaphoreType.DMA,
            pltpu.SemaphoreType.DMA,
        ])
    def k(h_hbm, w_hbm, src_hbm, dst_hbm, z_hbm, num_hbm,
          src_v, dst_v, w_v, idx0_v, idx1_v, rows0_v, rows1_v, acc_sh,
          semg0, semg1, sems):
        c = lax.axis_index("c")
        s = lax.axis_index("s")
        idxs = [idx0_v, idx1_v]
        rows = [rows0_v, rows1_v]
        semg = [semg0, semg1]

        def mkidx(kk, b, cbN):
            for t in range(8):
                idxs[b][pl.ds(t * L, L)] = src_v[kk, pl.ds(t * L, L)] + cbN

        def gather_start(b):
            pltpu.async_copy(h_hbm.at[idxs[b]], rows[b], semg[b])

        def gather_wait(b):
            pltpu.make_async_copy(h_hbm.at[idxs[b]], rows[b], semg[b]).wait()

        def scatter_drain(b):
            pltpu.make_async_copy(z_hbm.at[pl.ds(0, 128)], rows[b], sems).wait()

        def scale(kk, b):
            def body(jj, _):
                j = jj * 2
                for u in range(2):
                    ws = plsc.load_gather(
                        w_v, [jnp.full((L,), 0, i32) + (kk * 128 + j + u)])
                    for t in range(8):
                        sl = pl.ds(t * L, L)
                        rows[b][j + u, sl] = rows[b][j + u, sl] * ws
                return 0
            lax.fori_loop(0, 64, body, 0)

        for i in range(ncb):
            cb = i * NC + c if NB > 1 else 0
            hd = cb // bpb
            cbN = cb * NPAD
            pltpu.sync_copy(z_hbm, acc_sh.at[pl.ds(s * ZR, ZR)])
            plsc.subcore_barrier()

            for mm in range(nch):
                m = nch * s + mm if NB > 1 else c * NS + s
                pltpu.sync_copy(src_hbm.at[m], src_v)
                pltpu.sync_copy(dst_hbm.at[m], dst_v)
                pltpu.sync_copy(w_hbm.at[pl.ds((m * H + hd) * EW, EW)], w_v)

                mkidx(0, 0, cbN)
                gather_start(0)

                def sub2(kk2, _):
                    for b in range(2):
                        kk = kk2 * 2 + b
                        nb = 1 - b

                        @pl.when(kk + 1 < ER)
                        def _():
                            @pl.when(kk >= 1)
                            def _():
                                scatter_drain(nb)
                            mkidx(kk + 1, nb, cbN)
                            gather_start(nb)

                        gather_wait(b)
                        scale(kk, b)
                        pltpu.async_copy(rows[b], acc_sh.at[dst_v.at[kk]],
                                         sems, add=True)
                    return 0

                lax.fori_loop(0, ER // 2, sub2, 0)
                scatter_drain(0)
                scatter_drain(1)

            plsc.subcore_barrier()
            p = 0 if NB > 1 else c
            pltpu.sync_copy(acc_sh.at[pl.ds(s * ZR, ZR)],
                            num_hbm.at[p, cb, pl.ds(s * ZR, ZR)])

    return k(h_flat, w, srcR, dstR, zrow)


def _prep_edges(e):
    pad = EPAD - EFULL
    loop = jnp.arange(N, dtype=i32)
    fill = jnp.arange(pad, dtype=i32)
    src = jnp.concatenate([e[0], loop, fill % N])
    dst = jnp.concatenate([e[1], loop, N + (fill % 224)])
    return src.reshape(NW, ER, 128), dst.reshape(NW, ER, 128)


def _gat(x, epack, p, H, F, act):
    C = H * F
    NB = C // 128
    srcR, dstR = epack
    h = _matmul(x, p["W"])
    a_srcT, a_dstT = _att_scalars(h, p["att_src"], p["att_dst"], H)
    m_b = _softmax_shift(a_srcT, a_dstT, H)
    hbs = 1 if H == 1 else 2
    zrow_a = jnp.zeros((ZR,), f32)
    wps, dens = [], []
    for h0 in range(0, H, hbs):
        w_i, d_i = _pass_a(a_srcT[h0:h0 + hbs].reshape(-1),
                           a_dstT[h0:h0 + hbs].reshape(-1),
                           m_b[h0:h0 + hbs], srcR, dstR, zrow_a, hbs)
        wps.append(w_i.reshape(NW, hbs, EW))
        dens.append(d_i.reshape(NC, hbs, NPAD))
    w = wps[0] if len(wps) == 1 else jnp.concatenate(wps, 1)
    den = dens[0] if len(dens) == 1 else jnp.concatenate(dens, 1)
    den = jnp.transpose(den, (0, 2, 1))
    zrow_b = jnp.zeros((ZR, 128), f32)
    num = _pass_b(h.reshape(NB * NPAD, 128), w.reshape(-1), srcR, dstR,
                  zrow_b, F, NB, H)
    return _epilogue(h, num, den, p["bias"], H, F, act)


def kernel(x_gene, x_miRNA, x_TO, e_gene_gene, e_gene_pathway, e_gene_mirna,
           e_gene_TO, e_mirna_mirna, e_mirna_TO, e_TO_TO, params):
    egg = _prep_edges(e_gene_gene)
    egp = _prep_edges(e_gene_pathway)
    egm = _prep_edges(e_gene_mirna)
    egt = _prep_edges(e_gene_TO)
    emm = _prep_edges(e_mirna_mirna)
    emt = _prep_edges(e_mirna_TO)
    ett = _prep_edges(e_TO_TO)

    zp = ((0, NPAD - N), (0, 0))
    x_gene = jnp.pad(x_gene, zp)
    x_miRNA = jnp.pad(x_miRNA, zp)
    x_TO = jnp.pad(x_TO, zp)

    gx = _gat(x_gene, egg, params["gene_conv1"], 4, 256, True)
    gx = _gat(gx, egg, params["gene_conv2"], 1, 256, False)
    gx = _gat(gx, egp, params["gene_pathway_conv"], 1, 256, True)
    gx = _gat(gx, egm, params["gene_miRNA_conv"], 1, 256, True)
    gx = _gat(gx, egt, params["gene_TO_conv"], 1, 256, True)
    gx = _matmul(gx, params["fc_W"], params["fc_b"]).reshape(NPAD, 128)

    mx = _gat(x_miRNA, emm, params["miRNA_conv1"], 8, 128, True)
    mx = _gat(mx, emm, params["miRNA_conv2"], 1, 128, False)
    mx = _gat(mx, egm, params["miRNA_gene_conv"], 1, 128, True)
    mx = _gat(mx, emt, params["miRNA_TO_conv"], 1, 128, False)

    tx = _gat(x_TO, ett, params["TO_conv1"], 4, 256, True)
    tx = _gat(tx, ett, params["TO_conv2"], 1, 128, False)
    tx = _gat(tx, egt, params["TO_gene_conv"], 1, 128, True)
    tx = _gat(tx, emt, params["TO_miRNA_conv"], 1, 128, True)
    return (gx[:N], mx[:N], tx[:N])


# speedup vs baseline: 25.9000x; 1.0010x over previous
import functools

import jax
import jax.numpy as jnp
from jax import lax
from jax.experimental import pallas as pl
from jax.experimental.pallas import tpu as pltpu
from jax.experimental.pallas import tpu_sc as plsc

f32 = jnp.float32
i32 = jnp.int32

NC, NS, L = 2, 16, 16
NW = NC * NS
N = 10000
NPAD = 10240
E = 160000
EFULL = E + N
EPAD = NW * 5376
EROWS = EPAD // 128
BN = 1024


def _matmul(x, W, bias=None):
    Cin, C = W.shape
    NB = C // 128
    Wb = jnp.transpose(W.reshape(Cin, NB, 128), (1, 0, 2))

    if bias is None:
        def body(x_ref, w_ref, o_ref):
            o_ref[0] = jnp.dot(x_ref[...], w_ref[0], preferred_element_type=f32)
        in_specs = [
            pl.BlockSpec((BN, Cin), lambda cb, i: (i, 0)),
            pl.BlockSpec((1, Cin, 128), lambda cb, i: (cb, 0, 0)),
        ]
        args = (x, Wb)
    else:
        bb = bias.reshape(NB, 1, 128)

        def body(x_ref, w_ref, b_ref, o_ref):
            o_ref[0] = (jnp.dot(x_ref[...], w_ref[0], preferred_element_type=f32)
                        + b_ref[0])
        in_specs = [
            pl.BlockSpec((BN, Cin), lambda cb, i: (i, 0)),
            pl.BlockSpec((1, Cin, 128), lambda cb, i: (cb, 0, 0)),
            pl.BlockSpec((1, 1, 128), lambda cb, i: (cb, 0, 0)),
        ]
        args = (x, Wb, bb)

    return pl.pallas_call(
        body,
        grid=(NB, NPAD // BN),
        in_specs=in_specs,
        out_specs=pl.BlockSpec((1, BN, 128), lambda cb, i: (cb, i, 0)),
        out_shape=jax.ShapeDtypeStruct((NB, NPAD, 128), f32),
    )(*args)


def _att_scalars(h, att_src, att_dst, H):
    NB = h.shape[0]
    bpb = NB // H
    asv = att_src.reshape(NB, 1, 128)
    adv = att_dst.reshape(NB, 1, 128)

    def body(h_ref, as_ref, ad_ref, os_ref, od_ref):
        hb = h_ref[...]
        ts = (hb * as_ref[...]).sum(-1)
        td = (hb * ad_ref[...]).sum(-1)
        os_ref[...] = jnp.concatenate(
            [ts[hd * bpb:(hd + 1) * bpb].sum(0, keepdims=True) for hd in range(H)], 0)
        od_ref[...] = jnp.concatenate(
            [td[hd * bpb:(hd + 1) * bpb].sum(0, keepdims=True) for hd in range(H)], 0)

    return pl.pallas_call(
        body,
        grid=(NPAD // BN,),
        in_specs=[
            pl.BlockSpec((NB, BN, 128), lambda i: (0, i, 0)),
            pl.BlockSpec((NB, 1, 128), lambda i: (0, 0, 0)),
            pl.BlockSpec((NB, 1, 128), lambda i: (0, 0, 0)),
        ],
        out_specs=[
            pl.BlockSpec((H, BN), lambda i: (0, i)),
            pl.BlockSpec((H, BN), lambda i: (0, i)),
        ],
        out_shape=[
            jax.ShapeDtypeStruct((H, NPAD), f32),
            jax.ShapeDtypeStruct((H, NPAD), f32),
        ],
    )(h, asv, adv)


def _softmax_shift(a_srcT, a_dstT, H):
    def body(as_ref, ad_ref, m_ref):
        M = (jnp.max(as_ref[...], axis=1, keepdims=True)
             + jnp.max(ad_ref[...], axis=1, keepdims=True))
        M = jnp.maximum(M, 0.2 * M)
        m_ref[...] = jnp.broadcast_to(M, (H, 128))

    return pl.pallas_call(
        body,
        in_specs=[pl.BlockSpec((H, NPAD), lambda: (0, 0)),
                  pl.BlockSpec((H, NPAD), lambda: (0, 0))],
        out_specs=pl.BlockSpec((H, 128), lambda: (0, 0)),
        out_shape=jax.ShapeDtypeStruct((H, 128), f32),
    )(a_srcT, a_dstT)


def _epilogue(h, num, den, bias, H, F, act):
    NB = h.shape[0]
    P = num.shape[0]
    C = NB * 128
    bpb = F // 128
    bb = bias.reshape(1, C)

    def body(h_ref, num_ref, den_ref, b_ref, o_ref):
        i = pl.program_id(0)
        valid = (i * BN + lax.broadcasted_iota(i32, (BN, 128), 0)) < N
        dsum = den_ref[0] + den_ref[1]
        for cb in range(NB):
            hd = cb // bpb
            nm = num_ref[0, cb]
            for p in range(1, P):
                nm = nm + num_ref[p, cb]
            dn = dsum[:, hd:hd + 1] + 1e-16
            o = nm / dn + b_ref[:, cb * 128:(cb + 1) * 128]
            if act:
                o = jnp.where(o > 0, o, jnp.exp(jnp.minimum(o, 0.0)) - 1.0)
            o_ref[:, cb * 128:(cb + 1) * 128] = jnp.where(valid, o, 0.0)

    return pl.pallas_call(
        body,
        grid=(NPAD // BN,),
        in_specs=[
            pl.BlockSpec((NB, BN, 128), lambda i: (0, i, 0)),
            pl.BlockSpec((P, NB, BN, 128), lambda i: (0, 0, i, 0)),
            pl.BlockSpec((NC, BN, H), lambda i: (0, i, 0)),
            pl.BlockSpec((1, C), lambda i: (0, 0)),
        ],
        out_specs=pl.BlockSpec((BN, C), lambda i: (i, 0)),
        out_shape=jax.ShapeDtypeStruct((NPAD, C), f32),
    )(h, num, den, bb)


_MESH = plsc.VectorSubcoreMesh(core_axis_name="c", subcore_axis_name="s",
                               num_cores=NC, num_subcores=NS)

EW = EPAD // NW
ER = EW // 128
ZR = NPAD // NS


def _pass_a(a_srcF, a_dstF, m_b, srcR, dstR, zrow, hb):
    @functools.partial(
        pl.kernel,
        mesh=_MESH,
        compiler_params=pltpu.CompilerParams(use_tc_tiling_on_sc=False,
                                             needs_layout_passes=False),
        out_type=[jax.ShapeDtypeStruct((NW * hb * EW,), f32),
                  jax.ShapeDtypeStruct((NC * hb * NPAD,), f32)],
        scratch_types=[
            pltpu.VMEM((hb * NPAD,), f32),
            pltpu.VMEM((hb * NPAD,), f32),
            pltpu.VMEM((hb, 128), f32),
            pltpu.VMEM((ER, 128), i32),
            pltpu.VMEM((ER, 128), i32),
            pltpu.VMEM((hb * EW,), f32),
            pltpu.VMEM((128,), i32),
            pltpu.VMEM((128,), i32),
            pltpu.VMEM_SHARED((hb * NPAD,), f32),
        ])
    def k(asrc_hbm, adst_hbm, m_hbm, src_hbm, dst_hbm, z_hbm,
          w_hbm, den_hbm,
          asrc_v, adst_v, m_v, src_v, dst_v, wT_v, eix0_v, eix1_v, acc_sh):
        c = lax.axis_index("c")
        s = lax.axis_index("s")
        wid = c * NS + s
        eixs = [eix0_v, eix1_v]
        pltpu.sync_copy(asrc_hbm, asrc_v)
        pltpu.sync_copy(adst_hbm, adst_v)
        pltpu.sync_copy(m_hbm, m_v)
        pltpu.sync_copy(src_hbm.at[wid], src_v)
        pltpu.sync_copy(dst_hbm.at[wid], dst_v)
        for h in range(hb):
            pltpu.sync_copy(z_hbm, acc_sh.at[pl.ds(h * NPAD + s * ZR, ZR)])
        plsc.subcore_barrier()

        def sub(kk, _):
            for gg in range(8):
                sv = src_v[kk, pl.ds(gg * L, L)]
                dv = dst_v[kk, pl.ds(gg * L, L)]
                for h in range(hb):
                    av = plsc.load_gather(asrc_v, [sv + h * NPAD])
                    bv = plsc.load_gather(adst_v, [dv + h * NPAD])
                    e = av + bv
                    e = jnp.maximum(e, 0.2 * e)
                    w = jnp.exp(e - m_v[h, pl.ds(0, L)])
                    wT_v[pl.ds(h * EW + kk * 128 + gg * L, L)] = w
                    eixs[h][pl.ds(gg * L, L)] = dv + h * NPAD
            for h in range(hb):
                pltpu.sync_copy(wT_v.at[pl.ds(h * EW + kk * 128, 128)],
                                acc_sh.at[eixs[h]], add=True)
            return 0

        lax.fori_loop(0, ER, sub, 0)
        pltpu.sync_copy(wT_v, w_hbm.at[pl.ds(wid * hb * EW, hb * EW)])

        plsc.subcore_barrier()
        for h in range(hb):
            pltpu.sync_copy(
                acc_sh.at[pl.ds(h * NPAD + s * ZR, ZR)],
                den_hbm.at[pl.ds((c * hb + h) * NPAD + s * ZR, ZR)])

    return k(a_srcF, a_dstF, m_b, srcR, dstR, zrow)


def _pass_b(h_flat, w, srcR, dstR, zrow, F, NB, H):
    bpb = F // 128
    P = 1 if NB > 1 else NC
    ncb = NB // NC if NB > 1 else 1
    nch = 2 if NB > 1 else 1

    @functools.partial(
        pl.kernel,
        mesh=_MESH,
        compiler_params=pltpu.CompilerParams(use_tc_tiling_on_sc=False,
                                             needs_layout_passes=False),
        out_type=jax.ShapeDtypeStruct((P, NB, NPAD, 128), f32),
        scratch_types=[
            pltpu.VMEM((ER, 128), i32),
            pltpu.VMEM((ER, 128), i32),
            pltpu.VMEM((EW,), f32),
            pltpu.VMEM((128,), i32),
            pltpu.VMEM((128,), i32),
            pltpu.VMEM((128, 128), f32),
            pltpu.VMEM((128, 128), f32),
            pltpu.VMEM_SHARED((NPAD, 128), f32),
            pltpu.SemaphoreType.DMA,
            pltpu.SemaphoreType.DMA,
            pltpu.SemaphoreType.DMA,
        ])
    def k(h_hbm, w_hbm, src_hbm, dst_hbm, z_hbm, num_hbm,
          src_v, dst_v, w_v, idx0_v, idx1_v, rows0_v, rows1_v, acc_sh,
          semg0, semg1, sems):
        c = lax.axis_index("c")
        s = lax.axis_index("s")
        idxs = [idx0_v, idx1_v]
        rows = [rows0_v, rows1_v]
        semg = [semg0, semg1]

        def mkidx(kk, b, cbN):
            for t in range(8):
                idxs[b][pl.ds(t * L, L)] = src_v[kk, pl.ds(t * L, L)] + cbN

        def gather_start(b):
            pltpu.async_copy(h_hbm.at[idxs[b]], rows[b], semg[b])

        def gather_wait(b):
            pltpu.make_async_copy(h_hbm.at[idxs[b]], rows[b], semg[b]).wait()

        def scatter_drain(b):
            pltpu.make_async_copy(z_hbm.at[pl.ds(0, 128)], rows[b], sems).wait()

        def scale(kk, b):
            def body(jj, _):
                j = jj * 8
                for u in range(8):
                    ws = plsc.load_gather(
                        w_v, [jnp.full((L,), kk * 128 + j + u, i32)])
                    for t in range(8):
                        sl = pl.ds(t * L, L)
                        rows[b][j + u, sl] = rows[b][j + u, sl] * ws
                return 0
            lax.fori_loop(0, 16, body, 0)

        for i in range(ncb):
            cb = i * NC + c if NB > 1 else 0
            hd = cb // bpb
            cbN = cb * NPAD
            pltpu.sync_copy(z_hbm, acc_sh.at[pl.ds(s * ZR, ZR)])
            plsc.subcore_barrier()

            for mm in range(nch):
                m = nch * s + mm if NB > 1 else c * NS + s
                pltpu.sync_copy(src_hbm.at[m], src_v)
                pltpu.sync_copy(dst_hbm.at[m], dst_v)
                pltpu.sync_copy(w_hbm.at[pl.ds((m * H + hd) * EW, EW)], w_v)

                mkidx(0, 0, cbN)
                gather_start(0)

                def sub2(kk2, _):
                    for b in range(2):
                        kk = kk2 * 2 + b
                        nb = 1 - b

                        @pl.when(kk + 1 < ER)
                        def _():
                            @pl.when(kk >= 1)
                            def _():
                                scatter_drain(nb)
                            mkidx(kk + 1, nb, cbN)
                            gather_start(nb)

                        gather_wait(b)
                        scale(kk, b)
                        pltpu.async_copy(rows[b], acc_sh.at[dst_v.at[kk]],
                                         sems, add=True)
                    return 0

                lax.fori_loop(0, ER // 2, sub2, 0)
                scatter_drain(0)
                scatter_drain(1)

            plsc.subcore_barrier()
            p = 0 if NB > 1 else c
            pltpu.sync_copy(acc_sh.at[pl.ds(s * ZR, ZR)],
                            num_hbm.at[p, cb, pl.ds(s * ZR, ZR)])

    return k(h_flat, w, srcR, dstR, zrow)


def _prep_edges(e):
    pad = EPAD - EFULL
    loop = jnp.arange(N, dtype=i32)
    fill = jnp.arange(pad, dtype=i32)
    src = jnp.concatenate([e[0], loop, fill % N])
    dst = jnp.concatenate([e[1], loop, N + (fill % 224)])
    return src.reshape(NW, ER, 128), dst.reshape(NW, ER, 128)


def _gat(x, epack, p, H, F, act):
    C = H * F
    NB = C // 128
    srcR, dstR = epack
    h = _matmul(x, p["W"])
    a_srcT, a_dstT = _att_scalars(h, p["att_src"], p["att_dst"], H)
    m_b = _softmax_shift(a_srcT, a_dstT, H)
    hbs = 1 if H == 1 else 2
    zrow_a = jnp.zeros((ZR,), f32)
    wps, dens = [], []
    for h0 in range(0, H, hbs):
        w_i, d_i = _pass_a(a_srcT[h0:h0 + hbs].reshape(-1),
                           a_dstT[h0:h0 + hbs].reshape(-1),
                           m_b[h0:h0 + hbs], srcR, dstR, zrow_a, hbs)
        wps.append(w_i.reshape(NW, hbs, EW))
        dens.append(d_i.reshape(NC, hbs, NPAD))
    w = wps[0] if len(wps) == 1 else jnp.concatenate(wps, 1)
    den = dens[0] if len(dens) == 1 else jnp.concatenate(dens, 1)
    den = jnp.transpose(den, (0, 2, 1))
    zrow_b = jnp.zeros((ZR, 128), f32)
    num = _pass_b(h.reshape(NB * NPAD, 128), w.reshape(-1), srcR, dstR,
                  zrow_b, F, NB, H)
    return _epilogue(h, num, den, p["bias"], H, F, act)


def kernel(x_gene, x_miRNA, x_TO, e_gene_gene, e_gene_pathway, e_gene_mirna,
           e_gene_TO, e_mirna_mirna, e_mirna_TO, e_TO_TO, params):
    egg = _prep_edges(e_gene_gene)
    egp = _prep_edges(e_gene_pathway)
    egm = _prep_edges(e_gene_mirna)
    egt = _prep_edges(e_gene_TO)
    emm = _prep_edges(e_mirna_mirna)
    emt = _prep_edges(e_mirna_TO)
    ett = _prep_edges(e_TO_TO)

    zp = ((0, NPAD - N), (0, 0))
    x_gene = jnp.pad(x_gene, zp)
    x_miRNA = jnp.pad(x_miRNA, zp)
    x_TO = jnp.pad(x_TO, zp)

    gx = _gat(x_gene, egg, params["gene_conv1"], 4, 256, True)
    gx = _gat(gx, egg, params["gene_conv2"], 1, 256, False)
    gx = _gat(gx, egp, params["gene_pathway_conv"], 1, 256, True)
    gx = _gat(gx, egm, params["gene_miRNA_conv"], 1, 256, True)
    gx = _gat(gx, egt, params["gene_TO_conv"], 1, 256, True)
    gx = _matmul(gx, params["fc_W"], params["fc_b"]).reshape(NPAD, 128)

    mx = _gat(x_miRNA, emm, params["miRNA_conv1"], 8, 128, True)
    mx = _gat(mx, emm, params["miRNA_conv2"], 1, 128, False)
    mx = _gat(mx, egm, params["miRNA_gene_conv"], 1, 128, True)
    mx = _gat(mx, emt, params["miRNA_TO_conv"], 1, 128, False)

    tx = _gat(x_TO, ett, params["TO_conv1"], 4, 256, True)
    tx = _gat(tx, ett, params["TO_conv2"], 1, 128, False)
    tx = _gat(tx, egt, params["TO_gene_conv"], 1, 128, True)
    tx = _gat(tx, emt, params["TO_miRNA_conv"], 1, 128, True)
    return (gx[:N], mx[:N], tx[:N])


# speedup vs baseline: 27.6040x; 1.0658x over previous
import functools

import jax
import jax.numpy as jnp
from jax import lax
from jax.experimental import pallas as pl
from jax.experimental.pallas import tpu as pltpu
from jax.experimental.pallas import tpu_sc as plsc

f32 = jnp.float32
i32 = jnp.int32

NC, NS, L = 2, 16, 16
NW = NC * NS
N = 10000
NPAD = 10240
E = 160000
EFULL = E + N
EPAD = NW * 5376
EROWS = EPAD // 128
BN = 1024


def _matmul(x, W, bias=None):
    Cin, C = W.shape
    NB = C // 128
    Wb = jnp.transpose(W.reshape(Cin, NB, 128), (1, 0, 2))

    if bias is None:
        def body(x_ref, w_ref, o_ref):
            o_ref[0] = jnp.dot(x_ref[...], w_ref[0], preferred_element_type=f32)
        in_specs = [
            pl.BlockSpec((BN, Cin), lambda cb, i: (i, 0)),
            pl.BlockSpec((1, Cin, 128), lambda cb, i: (cb, 0, 0)),
        ]
        args = (x, Wb)
    else:
        bb = bias.reshape(NB, 1, 128)

        def body(x_ref, w_ref, b_ref, o_ref):
            o_ref[0] = (jnp.dot(x_ref[...], w_ref[0], preferred_element_type=f32)
                        + b_ref[0])
        in_specs = [
            pl.BlockSpec((BN, Cin), lambda cb, i: (i, 0)),
            pl.BlockSpec((1, Cin, 128), lambda cb, i: (cb, 0, 0)),
            pl.BlockSpec((1, 1, 128), lambda cb, i: (cb, 0, 0)),
        ]
        args = (x, Wb, bb)

    return pl.pallas_call(
        body,
        grid=(NB, NPAD // BN),
        in_specs=in_specs,
        out_specs=pl.BlockSpec((1, BN, 128), lambda cb, i: (cb, i, 0)),
        out_shape=jax.ShapeDtypeStruct((NB, NPAD, 128), f32),
    )(*args)


def _att_scalars(h, att_src, att_dst, H):
    NB = h.shape[0]
    bpb = NB // H
    asv = att_src.reshape(NB, 1, 128)
    adv = att_dst.reshape(NB, 1, 128)

    def body(h_ref, as_ref, ad_ref, os_ref, od_ref):
        hb = h_ref[...]
        ts = (hb * as_ref[...]).sum(-1)
        td = (hb * ad_ref[...]).sum(-1)
        os_ref[...] = jnp.concatenate(
            [ts[hd * bpb:(hd + 1) * bpb].sum(0, keepdims=True) for hd in range(H)], 0)
        od_ref[...] = jnp.concatenate(
            [td[hd * bpb:(hd + 1) * bpb].sum(0, keepdims=True) for hd in range(H)], 0)

    return pl.pallas_call(
        body,
        grid=(NPAD // BN,),
        in_specs=[
            pl.BlockSpec((NB, BN, 128), lambda i: (0, i, 0)),
            pl.BlockSpec((NB, 1, 128), lambda i: (0, 0, 0)),
            pl.BlockSpec((NB, 1, 128), lambda i: (0, 0, 0)),
        ],
        out_specs=[
            pl.BlockSpec((H, BN), lambda i: (0, i)),
            pl.BlockSpec((H, BN), lambda i: (0, i)),
        ],
        out_shape=[
            jax.ShapeDtypeStruct((H, NPAD), f32),
            jax.ShapeDtypeStruct((H, NPAD), f32),
        ],
    )(h, asv, adv)


def _softmax_shift(a_srcT, a_dstT, H):
    def body(as_ref, ad_ref, m_ref):
        M = (jnp.max(as_ref[...], axis=1, keepdims=True)
             + jnp.max(ad_ref[...], axis=1, keepdims=True))
        M = jnp.maximum(M, 0.2 * M)
        m_ref[...] = jnp.broadcast_to(M, (H, 128))

    return pl.pallas_call(
        body,
        in_specs=[pl.BlockSpec((H, NPAD), lambda: (0, 0)),
                  pl.BlockSpec((H, NPAD), lambda: (0, 0))],
        out_specs=pl.BlockSpec((H, 128), lambda: (0, 0)),
        out_shape=jax.ShapeDtypeStruct((H, 128), f32),
    )(a_srcT, a_dstT)


def _epilogue(h, num, den, bias, H, F, act):
    NB = h.shape[0]
    P = num.shape[0]
    C = NB * 128
    bpb = F // 128
    bb = bias.reshape(1, C)

    def body(h_ref, num_ref, den_ref, b_ref, o_ref):
        i = pl.program_id(0)
        valid = (i * BN + lax.broadcasted_iota(i32, (BN, 128), 0)) < N
        dsum = den_ref[0] + den_ref[1]
        for cb in range(NB):
            hd = cb // bpb
            nm = num_ref[0, cb]
            for p in range(1, P):
                nm = nm + num_ref[p, cb]
            dn = dsum[:, hd:hd + 1] + 1e-16
            o = nm / dn + b_ref[:, cb * 128:(cb + 1) * 128]
            if act:
                o = jnp.where(o > 0, o, jnp.exp(jnp.minimum(o, 0.0)) - 1.0)
            o_ref[:, cb * 128:(cb + 1) * 128] = jnp.where(valid, o, 0.0)

    return pl.pallas_call(
        body,
        grid=(NPAD // BN,),
        in_specs=[
            pl.BlockSpec((NB, BN, 128), lambda i: (0, i, 0)),
            pl.BlockSpec((P, NB, BN, 128), lambda i: (0, 0, i, 0)),
            pl.BlockSpec((NC, BN, H), lambda i: (0, i, 0)),
            pl.BlockSpec((1, C), lambda i: (0, 0)),
        ],
        out_specs=pl.BlockSpec((BN, C), lambda i: (i, 0)),
        out_shape=jax.ShapeDtypeStruct((NPAD, C), f32),
    )(h, num, den, bb)


_MESH = plsc.VectorSubcoreMesh(core_axis_name="c", subcore_axis_name="s",
                               num_cores=NC, num_subcores=NS)

EW = EPAD // NW
ER = EW // 128
SUB = 64
NSB = EW // SUB
ZR = NPAD // NS


def _pass_a(a_srcF, a_dstF, m_b, srcR, dstR, zrow, hb):
    @functools.partial(
        pl.kernel,
        mesh=_MESH,
        compiler_params=pltpu.CompilerParams(use_tc_tiling_on_sc=False,
                                             needs_layout_passes=False),
        out_type=[jax.ShapeDtypeStruct((NW * hb * EW,), f32),
                  jax.ShapeDtypeStruct((NC * hb * NPAD,), f32)],
        scratch_types=[
            pltpu.VMEM((hb * NPAD,), f32),
            pltpu.VMEM((hb * NPAD,), f32),
            pltpu.VMEM((hb, 128), f32),
            pltpu.VMEM((ER, 128), i32),
            pltpu.VMEM((ER, 128), i32),
            pltpu.VMEM((hb * EW,), f32),
            pltpu.VMEM((128,), i32),
            pltpu.VMEM((128,), i32),
            pltpu.VMEM_SHARED((hb * NPAD,), f32),
        ])
    def k(asrc_hbm, adst_hbm, m_hbm, src_hbm, dst_hbm, z_hbm,
          w_hbm, den_hbm,
          asrc_v, adst_v, m_v, src_v, dst_v, wT_v, eix0_v, eix1_v, acc_sh):
        c = lax.axis_index("c")
        s = lax.axis_index("s")
        wid = c * NS + s
        eixs = [eix0_v, eix1_v]
        pltpu.sync_copy(asrc_hbm, asrc_v)
        pltpu.sync_copy(adst_hbm, adst_v)
        pltpu.sync_copy(m_hbm, m_v)
        pltpu.sync_copy(src_hbm.at[wid], src_v)
        pltpu.sync_copy(dst_hbm.at[wid], dst_v)
        for h in range(hb):
            pltpu.sync_copy(z_hbm, acc_sh.at[pl.ds(h * NPAD + s * ZR, ZR)])
        plsc.subcore_barrier()

        def sub(kk, _):
            for gg in range(8):
                sv = src_v[kk, pl.ds(gg * L, L)]
                dv = dst_v[kk, pl.ds(gg * L, L)]
                for h in range(hb):
                    av = plsc.load_gather(asrc_v, [sv + h * NPAD])
                    bv = plsc.load_gather(adst_v, [dv + h * NPAD])
                    e = av + bv
                    e = jnp.maximum(e, 0.2 * e)
                    w = jnp.exp(e - m_v[h, pl.ds(0, L)])
                    wT_v[pl.ds(h * EW + kk * 128 + gg * L, L)] = w
                    eixs[h][pl.ds(gg * L, L)] = dv + h * NPAD
            for h in range(hb):
                pltpu.sync_copy(wT_v.at[pl.ds(h * EW + kk * 128, 128)],
                                acc_sh.at[eixs[h]], add=True)
            return 0

        lax.fori_loop(0, ER, sub, 0)
        pltpu.sync_copy(wT_v, w_hbm.at[pl.ds(wid * hb * EW, hb * EW)])

        plsc.subcore_barrier()
        for h in range(hb):
            pltpu.sync_copy(
                acc_sh.at[pl.ds(h * NPAD + s * ZR, ZR)],
                den_hbm.at[pl.ds((c * hb + h) * NPAD + s * ZR, ZR)])

    return k(a_srcF, a_dstF, m_b, srcR, dstR, zrow)


def _pass_b(h_flat, w, srcR, dstR, zrow, F, NB, H):
    bpb = F // 128
    P = 1 if NB > 1 else NC
    ncb = NB // NC if NB > 1 else 1
    nch = 2 if NB > 1 else 1

    @functools.partial(
        pl.kernel,
        mesh=_MESH,
        compiler_params=pltpu.CompilerParams(use_tc_tiling_on_sc=False,
                                             needs_layout_passes=False),
        out_type=jax.ShapeDtypeStruct((P, NB, NPAD, 128), f32),
        scratch_types=[
            pltpu.VMEM((NSB, SUB), i32),
            pltpu.VMEM((NSB, SUB), i32),
            pltpu.VMEM((EW,), f32),
            pltpu.VMEM((SUB,), i32),
            pltpu.VMEM((SUB,), i32),
            pltpu.VMEM((SUB,), i32),
            pltpu.VMEM((SUB, 128), f32),
            pltpu.VMEM((SUB, 128), f32),
            pltpu.VMEM((SUB, 128), f32),
            pltpu.VMEM_SHARED((NPAD, 128), f32),
            pltpu.SemaphoreType.DMA,
            pltpu.SemaphoreType.DMA,
            pltpu.SemaphoreType.DMA,
            pltpu.SemaphoreType.DMA,
        ])
    def k(h_hbm, w_hbm, src_hbm, dst_hbm, z_hbm, num_hbm,
          src_v, dst_v, w_v, idx0_v, idx1_v, idx2_v,
          rows0_v, rows1_v, rows2_v, acc_sh,
          semg0, semg1, semg2, sems):
        c = lax.axis_index("c")
        s = lax.axis_index("s")
        idxs = [idx0_v, idx1_v, idx2_v]
        rows = [rows0_v, rows1_v, rows2_v]
        semg = [semg0, semg1, semg2]

        def mkidx(kk, b, cbN):
            for t in range(SUB // L):
                idxs[b][pl.ds(t * L, L)] = src_v[kk, pl.ds(t * L, L)] + cbN

        def gather_start(b):
            pltpu.async_copy(h_hbm.at[idxs[b]], rows[b], semg[b])

        def gather_wait(b):
            pltpu.make_async_copy(h_hbm.at[idxs[b]], rows[b], semg[b]).wait()

        def scatter_drain(b):
            pltpu.make_async_copy(z_hbm.at[pl.ds(0, SUB)], rows[b], sems).wait()

        def scale(kk, b):
            def body(jj, _):
                j = jj * 8
                for u in range(8):
                    ws = plsc.load_gather(
                        w_v, [jnp.full((L,), kk * SUB + j + u, i32)])
                    for t in range(8):
                        sl = pl.ds(t * L, L)
                        rows[b][j + u, sl] = rows[b][j + u, sl] * ws
                return 0
            lax.fori_loop(0, SUB // 8, body, 0)

        for i in range(ncb):
            cb = i * NC + c if NB > 1 else 0
            hd = cb // bpb
            cbN = cb * NPAD
            pltpu.sync_copy(z_hbm, acc_sh.at[pl.ds(s * ZR, ZR)])
            plsc.subcore_barrier()

            for mm in range(nch):
                m = nch * s + mm if NB > 1 else c * NS + s
                pltpu.sync_copy(src_hbm.at[m], src_v)
                pltpu.sync_copy(dst_hbm.at[m], dst_v)
                pltpu.sync_copy(w_hbm.at[pl.ds((m * H + hd) * EW, EW)], w_v)

                mkidx(0, 0, cbN)
                gather_start(0)

                def sub3(kk3, _):
                    for b in range(3):
                        kk = kk3 * 3 + b
                        nb = (b + 1) % 3

                        @pl.when(kk + 1 < NSB)
                        def _():
                            @pl.when(kk >= 2)
                            def _():
                                scatter_drain(nb)
                            mkidx(kk + 1, nb, cbN)
                            gather_start(nb)

                        gather_wait(b)
                        scale(kk, b)
                        pltpu.async_copy(rows[b], acc_sh.at[dst_v.at[kk]],
                                         sems, add=True)
                    return 0

                lax.fori_loop(0, NSB // 3, sub3, 0)
                scatter_drain(0)
                scatter_drain(1)
                scatter_drain(2)

            plsc.subcore_barrier()
            p = 0 if NB > 1 else c
            pltpu.sync_copy(acc_sh.at[pl.ds(s * ZR, ZR)],
                            num_hbm.at[p, cb, pl.ds(s * ZR, ZR)])

    return k(h_flat, w, srcR, dstR, zrow)


def _prep_edges(e):
    pad = EPAD - EFULL
    loop = jnp.arange(N, dtype=i32)
    fill = jnp.arange(pad, dtype=i32)
    src = jnp.concatenate([e[0], loop, fill % N])
    dst = jnp.concatenate([e[1], loop, N + (fill % 224)])
    return src.reshape(NW, ER, 128), dst.reshape(NW, ER, 128)


def _gat(x, epack, p, H, F, act):
    C = H * F
    NB = C // 128
    srcR, dstR = epack
    h = _matmul(x, p["W"])
    a_srcT, a_dstT = _att_scalars(h, p["att_src"], p["att_dst"], H)
    m_b = _softmax_shift(a_srcT, a_dstT, H)
    hbs = 1 if H == 1 else 2
    zrow_a = jnp.zeros((ZR,), f32)
    wps, dens = [], []
    for h0 in range(0, H, hbs):
        w_i, d_i = _pass_a(a_srcT[h0:h0 + hbs].reshape(-1),
                           a_dstT[h0:h0 + hbs].reshape(-1),
                           m_b[h0:h0 + hbs], srcR, dstR, zrow_a, hbs)
        wps.append(w_i.reshape(NW, hbs, EW))
        dens.append(d_i.reshape(NC, hbs, NPAD))
    w = wps[0] if len(wps) == 1 else jnp.concatenate(wps, 1)
    den = dens[0] if len(dens) == 1 else jnp.concatenate(dens, 1)
    den = jnp.transpose(den, (0, 2, 1))
    zrow_b = jnp.zeros((ZR, 128), f32)
    num = _pass_b(h.reshape(NB * NPAD, 128), w.reshape(-1),
                  srcR.reshape(NW, NSB, SUB), dstR.reshape(NW, NSB, SUB),
                  zrow_b, F, NB, H)
    return _epilogue(h, num, den, p["bias"], H, F, act)


def kernel(x_gene, x_miRNA, x_TO, e_gene_gene, e_gene_pathway, e_gene_mirna,
           e_gene_TO, e_mirna_mirna, e_mirna_TO, e_TO_TO, params):
    egg = _prep_edges(e_gene_gene)
    egp = _prep_edges(e_gene_pathway)
    egm = _prep_edges(e_gene_mirna)
    egt = _prep_edges(e_gene_TO)
    emm = _prep_edges(e_mirna_mirna)
    emt = _prep_edges(e_mirna_TO)
    ett = _prep_edges(e_TO_TO)

    zp = ((0, NPAD - N), (0, 0))
    x_gene = jnp.pad(x_gene, zp)
    x_miRNA = jnp.pad(x_miRNA, zp)
    x_TO = jnp.pad(x_TO, zp)

    gx = _gat(x_gene, egg, params["gene_conv1"], 4, 256, True)
    gx = _gat(gx, egg, params["gene_conv2"], 1, 256, False)
    gx = _gat(gx, egp, params["gene_pathway_conv"], 1, 256, True)
    gx = _gat(gx, egm, params["gene_miRNA_conv"], 1, 256, True)
    gx = _gat(gx, egt, params["gene_TO_conv"], 1, 256, True)
    gx = _matmul(gx, params["fc_W"], params["fc_b"]).reshape(NPAD, 128)

    mx = _gat(x_miRNA, emm, params["miRNA_conv1"], 8, 128, True)
    mx = _gat(mx, emm, params["miRNA_conv2"], 1, 128, False)
    mx = _gat(mx, egm, params["miRNA_gene_conv"], 1, 128, True)
    mx = _gat(mx, emt, params["miRNA_TO_conv"], 1, 128, False)

    tx = _gat(x_TO, ett, params["TO_conv1"], 4, 256, True)
    tx = _gat(tx, ett, params["TO_conv2"], 1, 128, False)
    tx = _gat(tx, egt, params["TO_gene_conv"], 1, 128, True)
    tx = _gat(tx, emt, params["TO_miRNA_conv"], 1, 128, True)
    return (gx[:N], mx[:N], tx[:N])


# speedup vs baseline: 28.0984x; 1.0179x over previous
import functools

import jax
import jax.numpy as jnp
from jax import lax
from jax.experimental import pallas as pl
from jax.experimental.pallas import tpu as pltpu
from jax.experimental.pallas import tpu_sc as plsc

f32 = jnp.float32
i32 = jnp.int32

NC, NS, L = 2, 16, 16
NW = NC * NS
N = 10000
NPAD = 10240
E = 160000
EFULL = E + N
EPAD = NW * 5376
EROWS = EPAD // 128
BN = 1024


def _matmul(x, W, bias=None):
    Cin, C = W.shape
    NB = C // 128
    Wb = jnp.transpose(W.reshape(Cin, NB, 128), (1, 0, 2))

    if bias is None:
        def body(x_ref, w_ref, o_ref):
            o_ref[0] = jnp.dot(x_ref[...], w_ref[0], preferred_element_type=f32)
        in_specs = [
            pl.BlockSpec((BN, Cin), lambda cb, i: (i, 0)),
            pl.BlockSpec((1, Cin, 128), lambda cb, i: (cb, 0, 0)),
        ]
        args = (x, Wb)
    else:
        bb = bias.reshape(NB, 1, 128)

        def body(x_ref, w_ref, b_ref, o_ref):
            o_ref[0] = (jnp.dot(x_ref[...], w_ref[0], preferred_element_type=f32)
                        + b_ref[0])
        in_specs = [
            pl.BlockSpec((BN, Cin), lambda cb, i: (i, 0)),
            pl.BlockSpec((1, Cin, 128), lambda cb, i: (cb, 0, 0)),
            pl.BlockSpec((1, 1, 128), lambda cb, i: (cb, 0, 0)),
        ]
        args = (x, Wb, bb)

    return pl.pallas_call(
        body,
        grid=(NB, NPAD // BN),
        in_specs=in_specs,
        out_specs=pl.BlockSpec((1, BN, 128), lambda cb, i: (cb, i, 0)),
        out_shape=jax.ShapeDtypeStruct((NB, NPAD, 128), f32),
    )(*args)


def _att_scalars(h, att_src, att_dst, H):
    NB = h.shape[0]
    bpb = NB // H
    asv = att_src.reshape(NB, 1, 128)
    adv = att_dst.reshape(NB, 1, 128)

    def body(h_ref, as_ref, ad_ref, os_ref, od_ref):
        hb = h_ref[...]
        ts = (hb * as_ref[...]).sum(-1)
        td = (hb * ad_ref[...]).sum(-1)
        os_ref[...] = jnp.concatenate(
            [ts[hd * bpb:(hd + 1) * bpb].sum(0, keepdims=True) for hd in range(H)], 0)
        od_ref[...] = jnp.concatenate(
            [td[hd * bpb:(hd + 1) * bpb].sum(0, keepdims=True) for hd in range(H)], 0)

    return pl.pallas_call(
        body,
        grid=(NPAD // BN,),
        in_specs=[
            pl.BlockSpec((NB, BN, 128), lambda i: (0, i, 0)),
            pl.BlockSpec((NB, 1, 128), lambda i: (0, 0, 0)),
            pl.BlockSpec((NB, 1, 128), lambda i: (0, 0, 0)),
        ],
        out_specs=[
            pl.BlockSpec((H, BN), lambda i: (0, i)),
            pl.BlockSpec((H, BN), lambda i: (0, i)),
        ],
        out_shape=[
            jax.ShapeDtypeStruct((H, NPAD), f32),
            jax.ShapeDtypeStruct((H, NPAD), f32),
        ],
    )(h, asv, adv)


def _softmax_shift(a_srcT, a_dstT, H):
    def body(as_ref, ad_ref, m_ref):
        M = (jnp.max(as_ref[...], axis=1, keepdims=True)
             + jnp.max(ad_ref[...], axis=1, keepdims=True))
        M = jnp.maximum(M, 0.2 * M)
        m_ref[...] = jnp.broadcast_to(M, (H, 128))

    return pl.pallas_call(
        body,
        in_specs=[pl.BlockSpec((H, NPAD), lambda: (0, 0)),
                  pl.BlockSpec((H, NPAD), lambda: (0, 0))],
        out_specs=pl.BlockSpec((H, 128), lambda: (0, 0)),
        out_shape=jax.ShapeDtypeStruct((H, 128), f32),
    )(a_srcT, a_dstT)


def _epilogue(h, num, den, bias, H, F, act):
    NB = h.shape[0]
    P = num.shape[0]
    C = NB * 128
    bpb = F // 128
    bb = bias.reshape(1, C)

    def body(h_ref, num_ref, den_ref, b_ref, o_ref):
        i = pl.program_id(0)
        valid = (i * BN + lax.broadcasted_iota(i32, (BN, 128), 0)) < N
        dsum = den_ref[0] + den_ref[1]
        for cb in range(NB):
            hd = cb // bpb
            nm = num_ref[0, cb]
            for p in range(1, P):
                nm = nm + num_ref[p, cb]
            dn = dsum[:, hd:hd + 1] + 1e-16
            o = nm / dn + b_ref[:, cb * 128:(cb + 1) * 128]
            if act:
                o = jnp.where(o > 0, o, jnp.exp(jnp.minimum(o, 0.0)) - 1.0)
            o_ref[:, cb * 128:(cb + 1) * 128] = jnp.where(valid, o, 0.0)

    return pl.pallas_call(
        body,
        grid=(NPAD // BN,),
        in_specs=[
            pl.BlockSpec((NB, BN, 128), lambda i: (0, i, 0)),
            pl.BlockSpec((P, NB, BN, 128), lambda i: (0, 0, i, 0)),
            pl.BlockSpec((NC, BN, H), lambda i: (0, i, 0)),
            pl.BlockSpec((1, C), lambda i: (0, 0)),
        ],
        out_specs=pl.BlockSpec((BN, C), lambda i: (i, 0)),
        out_shape=jax.ShapeDtypeStruct((NPAD, C), f32),
    )(h, num, den, bb)


_MESH = plsc.VectorSubcoreMesh(core_axis_name="c", subcore_axis_name="s",
                               num_cores=NC, num_subcores=NS)

EW = EPAD // NW
ER = EW // 128
SUB = 64
NSB = EW // SUB
ZR = NPAD // NS


def _pass_a(a_srcF, a_dstF, m_b, srcR, dstR, zrow, hb):
    @functools.partial(
        pl.kernel,
        mesh=_MESH,
        compiler_params=pltpu.CompilerParams(use_tc_tiling_on_sc=False,
                                             needs_layout_passes=False),
        out_type=[jax.ShapeDtypeStruct((NW * hb * EW,), f32),
                  jax.ShapeDtypeStruct((NC * hb * NPAD,), f32)],
        scratch_types=[
            pltpu.VMEM((hb * NPAD,), f32),
            pltpu.VMEM((hb * NPAD,), f32),
            pltpu.VMEM((hb, 128), f32),
            pltpu.VMEM((ER, 128), i32),
            pltpu.VMEM((ER, 128), i32),
            pltpu.VMEM((hb * EW,), f32),
            pltpu.VMEM((3, 2, 128), i32),
            pltpu.VMEM((128,), f32),
            pltpu.VMEM_SHARED((hb * NPAD,), f32),
            pltpu.SemaphoreType.DMA,
        ])
    def k(asrc_hbm, adst_hbm, m_hbm, src_hbm, dst_hbm, z_hbm,
          w_hbm, den_hbm,
          asrc_v, adst_v, m_v, src_v, dst_v, wT_v, eix_v, drain_v, acc_sh,
          sema):
        c = lax.axis_index("c")
        s = lax.axis_index("s")
        wid = c * NS + s

        def denom_drain():
            for _h in range(hb):
                pltpu.make_async_copy(z_hbm.at[pl.ds(0, 128)], drain_v,
                                      sema).wait()
        pltpu.sync_copy(asrc_hbm, asrc_v)
        pltpu.sync_copy(adst_hbm, adst_v)
        pltpu.sync_copy(m_hbm, m_v)
        pltpu.sync_copy(src_hbm.at[wid], src_v)
        pltpu.sync_copy(dst_hbm.at[wid], dst_v)
        for h in range(hb):
            pltpu.sync_copy(z_hbm, acc_sh.at[pl.ds(h * NPAD + s * ZR, ZR)])
        plsc.subcore_barrier()

        def sub3(kk3, _):
            for b in range(3):
                kk = kk3 * 3 + b

                @pl.when(kk >= 3)
                def _():
                    denom_drain()

                for gg in range(8):
                    sv = src_v[kk, pl.ds(gg * L, L)]
                    dv = dst_v[kk, pl.ds(gg * L, L)]
                    for h in range(hb):
                        av = plsc.load_gather(asrc_v, [sv + h * NPAD])
                        bv = plsc.load_gather(adst_v, [dv + h * NPAD])
                        e = av + bv
                        e = jnp.maximum(e, 0.2 * e)
                        w = jnp.exp(e - m_v[h, pl.ds(0, L)])
                        wT_v[pl.ds(h * EW + kk * 128 + gg * L, L)] = w
                        eix_v[b, h, pl.ds(gg * L, L)] = dv + h * NPAD
                for h in range(hb):
                    pltpu.async_copy(wT_v.at[pl.ds(h * EW + kk * 128, 128)],
                                     acc_sh.at[eix_v.at[b, h]], sema, add=True)
            return 0

        lax.fori_loop(0, ER // 3, sub3, 0)
        denom_drain()
        denom_drain()
        denom_drain()
        pltpu.sync_copy(wT_v, w_hbm.at[pl.ds(wid * hb * EW, hb * EW)])

        plsc.subcore_barrier()
        for h in range(hb):
            pltpu.sync_copy(
                acc_sh.at[pl.ds(h * NPAD + s * ZR, ZR)],
                den_hbm.at[pl.ds((c * hb + h) * NPAD + s * ZR, ZR)])

    return k(a_srcF, a_dstF, m_b, srcR, dstR, zrow)


def _pass_b(h_flat, w, srcR, dstR, zrow, F, NB, H):
    bpb = F // 128
    P = 1 if NB > 1 else NC
    ncb = NB // NC if NB > 1 else 1
    nch = 2 if NB > 1 else 1

    @functools.partial(
        pl.kernel,
        mesh=_MESH,
        compiler_params=pltpu.CompilerParams(use_tc_tiling_on_sc=False,
                                             needs_layout_passes=False),
        out_type=jax.ShapeDtypeStruct((P, NB, NPAD, 128), f32),
        scratch_types=[
            pltpu.VMEM((NSB, SUB), i32),
            pltpu.VMEM((NSB, SUB), i32),
            pltpu.VMEM((EW,), f32),
            pltpu.VMEM((SUB,), i32),
            pltpu.VMEM((SUB,), i32),
            pltpu.VMEM((SUB,), i32),
            pltpu.VMEM((SUB, 128), f32),
            pltpu.VMEM((SUB, 128), f32),
            pltpu.VMEM((SUB, 128), f32),
            pltpu.VMEM_SHARED((NPAD, 128), f32),
            pltpu.SemaphoreType.DMA,
            pltpu.SemaphoreType.DMA,
            pltpu.SemaphoreType.DMA,
            pltpu.SemaphoreType.DMA,
        ])
    def k(h_hbm, w_hbm, src_hbm, dst_hbm, z_hbm, num_hbm,
          src_v, dst_v, w_v, idx0_v, idx1_v, idx2_v,
          rows0_v, rows1_v, rows2_v, acc_sh,
          semg0, semg1, semg2, sems):
        c = lax.axis_index("c")
        s = lax.axis_index("s")
        idxs = [idx0_v, idx1_v, idx2_v]
        rows = [rows0_v, rows1_v, rows2_v]
        semg = [semg0, semg1, semg2]

        def mkidx(kk, b, cbN):
            for t in range(SUB // L):
                idxs[b][pl.ds(t * L, L)] = src_v[kk, pl.ds(t * L, L)] + cbN

        def gather_start(b):
            pltpu.async_copy(h_hbm.at[idxs[b]], rows[b], semg[b])

        def gather_wait(b):
            pltpu.make_async_copy(h_hbm.at[idxs[b]], rows[b], semg[b]).wait()

        def scatter_drain(b):
            pltpu.make_async_copy(z_hbm.at[pl.ds(0, SUB)], rows[b], sems).wait()

        def scale(kk, b):
            def body(jj, _):
                j = jj * 8
                for u in range(8):
                    ws = plsc.load_gather(
                        w_v, [jnp.full((L,), kk * SUB + j + u, i32)])
                    for t in range(8):
                        sl = pl.ds(t * L, L)
                        rows[b][j + u, sl] = rows[b][j + u, sl] * ws
                return 0
            lax.fori_loop(0, SUB // 8, body, 0)

        for i in range(ncb):
            cb = i * NC + c if NB > 1 else 0
            hd = cb // bpb
            cbN = cb * NPAD
            pltpu.sync_copy(z_hbm, acc_sh.at[pl.ds(s * ZR, ZR)])
            plsc.subcore_barrier()

            for mm in range(nch):
                m = nch * s + mm if NB > 1 else c * NS + s
                pltpu.sync_copy(src_hbm.at[m], src_v)
                pltpu.sync_copy(dst_hbm.at[m], dst_v)
                pltpu.sync_copy(w_hbm.at[pl.ds((m * H + hd) * EW, EW)], w_v)

                mkidx(0, 0, cbN)
                gather_start(0)

                def sub3(kk3, _):
                    for b in range(3):
                        kk = kk3 * 3 + b
                        nb = (b + 1) % 3

                        @pl.when(kk + 1 < NSB)
                        def _():
                            @pl.when(kk >= 2)
                            def _():
                                scatter_drain(nb)
                            mkidx(kk + 1, nb, cbN)
                            gather_start(nb)

                        gather_wait(b)
                        scale(kk, b)
                        pltpu.async_copy(rows[b], acc_sh.at[dst_v.at[kk]],
                                         sems, add=True)
                    return 0

                lax.fori_loop(0, NSB // 3, sub3, 0)
                scatter_drain(0)
                scatter_drain(1)
                scatter_drain(2)

            plsc.subcore_barrier()
            p = 0 if NB > 1 else c
            pltpu.sync_copy(acc_sh.at[pl.ds(s * ZR, ZR)],
                            num_hbm.at[p, cb, pl.ds(s * ZR, ZR)])

    return k(h_flat, w, srcR, dstR, zrow)


def _prep_edges(e):
    pad = EPAD - EFULL
    loop = jnp.arange(N, dtype=i32)
    fill = jnp.arange(pad, dtype=i32)
    src = jnp.concatenate([e[0], loop, fill % N])
    dst = jnp.concatenate([e[1], loop, N + (fill % 224)])
    return src.reshape(NW, ER, 128), dst.reshape(NW, ER, 128)


def _gat(x, epack, p, H, F, act):
    C = H * F
    NB = C // 128
    srcR, dstR = epack
    h = _matmul(x, p["W"])
    a_srcT, a_dstT = _att_scalars(h, p["att_src"], p["att_dst"], H)
    m_b = _softmax_shift(a_srcT, a_dstT, H)
    hbs = 1 if H == 1 else 2
    zrow_a = jnp.zeros((ZR,), f32)
    wps, dens = [], []
    for h0 in range(0, H, hbs):
        w_i, d_i = _pass_a(a_srcT[h0:h0 + hbs].reshape(-1),
                           a_dstT[h0:h0 + hbs].reshape(-1),
                           m_b[h0:h0 + hbs], srcR, dstR, zrow_a, hbs)
        wps.append(w_i.reshape(NW, hbs, EW))
        dens.append(d_i.reshape(NC, hbs, NPAD))
    w = wps[0] if len(wps) == 1 else jnp.concatenate(wps, 1)
    den = dens[0] if len(dens) == 1 else jnp.concatenate(dens, 1)
    den = jnp.transpose(den, (0, 2, 1))
    zrow_b = jnp.zeros((ZR, 128), f32)
    num = _pass_b(h.reshape(NB * NPAD, 128), w.reshape(-1),
                  srcR.reshape(NW, NSB, SUB), dstR.reshape(NW, NSB, SUB),
                  zrow_b, F, NB, H)
    return _epilogue(h, num, den, p["bias"], H, F, act)


def kernel(x_gene, x_miRNA, x_TO, e_gene_gene, e_gene_pathway, e_gene_mirna,
           e_gene_TO, e_mirna_mirna, e_mirna_TO, e_TO_TO, params):
    egg = _prep_edges(e_gene_gene)
    egp = _prep_edges(e_gene_pathway)
    egm = _prep_edges(e_gene_mirna)
    egt = _prep_edges(e_gene_TO)
    emm = _prep_edges(e_mirna_mirna)
    emt = _prep_edges(e_mirna_TO)
    ett = _prep_edges(e_TO_TO)

    zp = ((0, NPAD - N), (0, 0))
    x_gene = jnp.pad(x_gene, zp)
    x_miRNA = jnp.pad(x_miRNA, zp)
    x_TO = jnp.pad(x_TO, zp)

    gx = _gat(x_gene, egg, params["gene_conv1"], 4, 256, True)
    gx = _gat(gx, egg, params["gene_conv2"], 1, 256, False)
    gx = _gat(gx, egp, params["gene_pathway_conv"], 1, 256, True)
    gx = _gat(gx, egm, params["gene_miRNA_conv"], 1, 256, True)
    gx = _gat(gx, egt, params["gene_TO_conv"], 1, 256, True)
    gx = _matmul(gx, params["fc_W"], params["fc_b"]).reshape(NPAD, 128)

    mx = _gat(x_miRNA, emm, params["miRNA_conv1"], 8, 128, True)
    mx = _gat(mx, emm, params["miRNA_conv2"], 1, 128, False)
    mx = _gat(mx, egm, params["miRNA_gene_conv"], 1, 128, True)
    mx = _gat(mx, emt, params["miRNA_TO_conv"], 1, 128, False)

    tx = _gat(x_TO, ett, params["TO_conv1"], 4, 256, True)
    tx = _gat(tx, ett, params["TO_conv2"], 1, 128, False)
    tx = _gat(tx, egt, params["TO_gene_conv"], 1, 128, True)
    tx = _gat(tx, emt, params["TO_miRNA_conv"], 1, 128, True)
    return (gx[:N], mx[:N], tx[:N])
